# baseline (speedup 1.0000x reference)
import functools

import jax
import jax.numpy as jnp
from jax import lax
from jax.experimental import pallas as pl
from jax.experimental.pallas import tpu as pltpu
from jax.experimental.pallas import tpu_sc as plsc

_NUM_CORES = 2
_NUM_SUBCORES = 16
_NUM_WORKERS = _NUM_CORES * _NUM_SUBCORES

_BT = 128
_NBUF = 2


@functools.lru_cache(maxsize=None)
def _make_gather(bsz, seq, v, d):
    assert bsz % (_BT * _NUM_WORKERS) == 0 or bsz == _BT * _NUM_WORKERS
    assert d % 16 == 0 and seq % _NBUF == 0
    wide = 2 * d
    n = bsz * seq

    mesh = plsc.VectorSubcoreMesh(
        core_axis_name="c", subcore_axis_name="s", num_cores=_NUM_CORES
    )

    @functools.partial(
        pl.kernel,
        out_type=jax.ShapeDtypeStruct((seq, d, bsz), jnp.float32),
        mesh=mesh,
        scratch_types=[
            pltpu.VMEM((_BT * seq // wide, wide), jnp.int32),
            pltpu.VMEM((seq, _BT), jnp.int32),
            tuple(pltpu.VMEM((_BT,), jnp.int32) for _ in range(_NBUF)),
            tuple(pltpu.VMEM((_BT, wide), jnp.float32) for _ in range(_NBUF)),
            tuple(pltpu.VMEM((d, _BT), jnp.float32) for _ in range(_NBUF)),
            tuple(pltpu.SemaphoreType.DMA for _ in range(_NBUF)),
            tuple(pltpu.SemaphoreType.DMA for _ in range(_NBUF)),
        ],
        compiler_params=pltpu.CompilerParams(needs_layout_passes=False),
    )
    def gather_kernel(idx_hbm, table_hbm, out_hbm, blk, idxT, widx, rows,
                      obuf, gsems, wsems):
        wid = lax.axis_index("s") * _NUM_CORES + lax.axis_index("c")
        blk_rows = _BT * seq // wide
        pltpu.sync_copy(idx_hbm.at[pl.ds(wid * blk_rows, blk_rows)], blk)

        iota = lax.iota(jnp.int32, 16)

        def t_body(s, carry):
            for l in range(_BT // 16):
                flat = (iota + (16 * l)) * seq + s
                tok16 = plsc.load_gather(
                    blk, [lax.shift_right_logical(flat, 7), flat & 127])
                idxT[s, pl.ds(16 * l, 16)] = tok16
            return carry

        lax.fori_loop(0, seq, t_body, 0)

        def g_start(s, b):
            for l in range(_BT // 16):
                tok16 = idxT[s, pl.ds(16 * l, 16)]
                widx[b][pl.ds(16 * l, 16)] = lax.shift_right_logical(tok16, 1)
            pltpu.async_copy(table_hbm.at[widx[b]], rows[b], gsems[b])

        def g_wait(b):
            pltpu.make_async_copy(
                table_hbm.at[widx[b]], rows[b], gsems[b]).wait()

        def build(s, b):
            for l in range(_BT // 16):
                tok16 = idxT[s, pl.ds(16 * l, 16)]
                off16 = (tok16 & 1) * d
                row16 = iota + (16 * l)
                for e in range(d):
                    val = plsc.load_gather(rows[b], [row16, off16 + e])
                    obuf[b][e, pl.ds(16 * l, 16)] = val

        def w_start(s, b):
            pltpu.async_copy(
                obuf[b],
                out_hbm.at[s, :, pl.ds(wid * _BT, _BT)],
                wsems[b])

        def w_wait(b):
            pltpu.make_async_copy(
                obuf[b], out_hbm.at[0, :, pl.ds(wid * _BT, _BT)],
                wsems[b]).wait()

        for j in range(_NBUF - 1):
            g_start(j, j)

        def step(s, b):
            g_wait(b)
            pb = (b - 1) % _NBUF

            @pl.when(s >= 1)
            def _():
                w_wait(pb)

            @pl.when(s + _NBUF - 1 < seq)
            def _():
                g_start(s + _NBUF - 1, pb)

            build(s, b)
            w_start(s, b)

        def outer(i, carry):
            for b in range(_NBUF):
                step(i * _NBUF + b, b)
            return carry

        lax.fori_loop(0, seq // _NBUF, outer, 0)
        w_wait((seq - 1) % _NBUF)

    return gather_kernel


def kernel(tokens, table):
    bsz, seq = tokens.shape
    v, d = table.shape
    idx = tokens.reshape(bsz * seq // (2 * d), 2 * d).astype(jnp.int32)
    out = _make_gather(bsz, seq, v, d)(idx, table.reshape(v // 2, 2 * d))
    return out.transpose(2, 0, 1)


# speedup vs baseline: 1.2621x; 1.2621x over previous
import functools

import jax
import jax.numpy as jnp
from jax import lax
from jax.experimental import pallas as pl
from jax.experimental.pallas import tpu as pltpu
from jax.experimental.pallas import tpu_sc as plsc

_NUM_CORES = 2
_NUM_SUBCORES = 16
_NUM_WORKERS = _NUM_CORES * _NUM_SUBCORES

_BT = 128
_NBUF = 2
_OSTRIDE = _BT + 1


@functools.lru_cache(maxsize=None)
def _make_gather(bsz, seq, v, d):
    assert bsz == _BT * _NUM_WORKERS and d % 16 == 0 and seq % _NBUF == 0
    wide = 2 * d

    mesh = plsc.VectorSubcoreMesh(
        core_axis_name="c", subcore_axis_name="s", num_cores=_NUM_CORES
    )

    @functools.partial(
        pl.kernel,
        out_type=jax.ShapeDtypeStruct((seq, d, bsz), jnp.float32),
        mesh=mesh,
        scratch_types=[
            pltpu.VMEM((_BT * seq // wide, wide), jnp.int32),
            pltpu.VMEM((seq, _BT), jnp.int32),
            tuple(pltpu.VMEM((_BT, wide), jnp.float32) for _ in range(_NBUF)),
            tuple(pltpu.VMEM((d, _OSTRIDE), jnp.float32) for _ in range(_NBUF)),
            tuple(pltpu.SemaphoreType.DMA for _ in range(_NBUF)),
            tuple(pltpu.SemaphoreType.DMA for _ in range(_NBUF)),
        ],
        compiler_params=pltpu.CompilerParams(needs_layout_passes=False),
    )
    def gather_kernel(idx_hbm, table_hbm, out_hbm, blk, idxT, rows, obuf,
                      gsems, wsems):
        wid = lax.axis_index("s") * _NUM_CORES + lax.axis_index("c")
        blk_rows = _BT * seq // wide
        pltpu.sync_copy(idx_hbm.at[pl.ds(wid * blk_rows, blk_rows)], blk)

        iota = lax.iota(jnp.int32, 16)

        def t_body(s, carry):
            for l in range(_BT // 16):
                flat = (iota + (16 * l)) * seq + s
                tok16 = plsc.load_gather(
                    blk, [lax.shift_right_logical(flat, 7), flat & 127])
                idxT[s, pl.ds(16 * l, 16)] = tok16
            return carry

        lax.fori_loop(0, seq, t_body, 0)

        def g_start(s, b):
            pltpu.async_copy(table_hbm.at[idxT.at[s]], rows[b], gsems[b])

        def g_wait(b):
            pltpu.make_async_copy(
                table_hbm.at[idxT.at[0]], rows[b], gsems[b]).wait()

        obase = [iota + (16 * k) for k in range(d // 16)]

        def build(b):
            def bi_body(ig, carry):
                for j in range(8):
                    i = ig * 8 + j
                    col = jnp.full((16,), i, jnp.int32)
                    for k in range(d // 16):
                        val = rows[b][i, pl.ds(16 * k, 16)]
                        plsc.store_scatter(obuf[b], [obase[k], col], val)
                return carry

            lax.fori_loop(0, _BT // 8, bi_body, 0)

        def w_start(s, b):
            pltpu.async_copy(
                obuf[b].at[:, pl.ds(0, _BT)],
                out_hbm.at[s, :, pl.ds(wid * _BT, _BT)],
                wsems[b])

        def w_wait(b):
            pltpu.make_async_copy(
                obuf[b].at[:, pl.ds(0, _BT)],
                out_hbm.at[0, :, pl.ds(wid * _BT, _BT)],
                wsems[b]).wait()

        for j in range(_NBUF - 1):
            g_start(j, j)

        def step(s, b):
            g_wait(b)
            pb = (b - 1) % _NBUF

            @pl.when(s >= 1)
            def _():
                w_wait(pb)

            @pl.when(s + _NBUF - 1 < seq)
            def _():
                g_start(s + _NBUF - 1, pb)

            build(b)
            w_start(s, b)

        def outer(i, carry):
            for b in range(_NBUF):
                step(i * _NBUF + b, b)
            return carry

        lax.fori_loop(0, seq // _NBUF, outer, 0)
        w_wait((seq - 1) % _NBUF)

    return gather_kernel


def kernel(tokens, table):
    bsz, seq = tokens.shape
    v, d = table.shape
    idx = tokens.reshape(bsz * seq // (2 * d), 2 * d).astype(jnp.int32)
    table_w = jnp.pad(table, ((0, 0), (0, d)))
    out = _make_gather(bsz, seq, v, d)(idx, table_w)
    return out.transpose(2, 0, 1)


# speedup vs baseline: 2.0009x; 1.5854x over previous
import functools

import jax
import jax.numpy as jnp
from jax import lax
from jax.experimental import pallas as pl
from jax.experimental.pallas import tpu as pltpu
from jax.experimental.pallas import tpu_sc as plsc

_NUM_CORES = 2
_NUM_SUBCORES = 16
_NUM_WORKERS = _NUM_CORES * _NUM_SUBCORES

_CHUNK = 256
_NBUF = 2


@functools.lru_cache(maxsize=None)
def _make_gather(n, v, d):
    n_per_w = n // _NUM_WORKERS
    n_chunks = n_per_w // _CHUNK
    assert n_per_w * _NUM_WORKERS == n and n_chunks * _CHUNK == n_per_w
    assert n_chunks % _NBUF == 0
    wide = 2 * d

    mesh = plsc.VectorSubcoreMesh(
        core_axis_name="c", subcore_axis_name="s", num_cores=_NUM_CORES
    )

    @functools.partial(
        pl.kernel,
        out_type=jax.ShapeDtypeStruct((n, wide), jnp.float32),
        mesh=mesh,
        scratch_types=[
            pltpu.VMEM((n_per_w,), jnp.int32),
            tuple(pltpu.VMEM((_CHUNK, wide), jnp.float32) for _ in range(_NBUF)),
            tuple(pltpu.SemaphoreType.DMA for _ in range(_NBUF)),
            tuple(pltpu.SemaphoreType.DMA for _ in range(_NBUF)),
        ],
        compiler_params=pltpu.CompilerParams(needs_layout_passes=False),
    )
    def gather_kernel(idx_hbm, table_hbm, out_hbm, idx_all, rows, gsems, wsems):
        wid = lax.axis_index("s") * _NUM_CORES + lax.axis_index("c")
        base = wid * n_per_w

        pltpu.sync_copy(idx_hbm.at[pl.ds(base, n_per_w)], idx_all)

        def g_start(g, b):
            pltpu.async_copy(
                table_hbm.at[idx_all.at[pl.ds(g * _CHUNK, _CHUNK)]],
                rows[b], gsems[b])

        def g_wait(b):
            pltpu.make_async_copy(
                table_hbm.at[idx_all.at[pl.ds(0, _CHUNK)]],
                rows[b], gsems[b]).wait()

        def w_start(g, b):
            pltpu.async_copy(
                rows[b], out_hbm.at[pl.ds(base + g * _CHUNK, _CHUNK)],
                wsems[b])

        def w_wait(b):
            pltpu.make_async_copy(
                rows[b], out_hbm.at[pl.ds(base, _CHUNK)], wsems[b]).wait()

        for j in range(_NBUF - 1):
            g_start(j, j)

        def step(g, b):
            g_wait(b)
            pb = (b - 1) % _NBUF

            @pl.when(g >= 1)
            def _():
                w_wait(pb)

            @pl.when(g + _NBUF - 1 < n_chunks)
            def _():
                g_start(g + _NBUF - 1, pb)

            w_start(g, b)

        def outer(i, carry):
            for b in range(_NBUF):
                step(i * _NBUF + b, b)
            return carry

        lax.fori_loop(0, n_chunks // _NBUF, outer, 0)
        w_wait((n_chunks - 1) % _NBUF)

    return gather_kernel


def kernel(tokens, table):
    b, s = tokens.shape
    v, d = table.shape
    n = b * s
    idx = tokens.reshape(n).astype(jnp.int32)
    table_w = jnp.pad(table, ((0, 0), (0, d)))
    wide_out = _make_gather(n, v, d)(idx, table_w)
    return lax.slice(wide_out, (0, 0), (n, d)).reshape(b, s, d)


# speedup vs baseline: 2.0090x; 1.0041x over previous
import functools

import jax
import jax.numpy as jnp
from jax import lax
from jax.experimental import pallas as pl
from jax.experimental.pallas import tpu as pltpu
from jax.experimental.pallas import tpu_sc as plsc

_NUM_CORES = 2
_NUM_SUBCORES = 16
_NUM_WORKERS = _NUM_CORES * _NUM_SUBCORES

_CHUNK = 128
_NBUF = 4


@functools.lru_cache(maxsize=None)
def _make_gather(n, v, d):
    n_per_w = n // _NUM_WORKERS
    n_chunks = n_per_w // _CHUNK
    assert n_per_w * _NUM_WORKERS == n and n_chunks * _CHUNK == n_per_w
    assert n_chunks % _NBUF == 0
    wide = 2 * d

    mesh = plsc.VectorSubcoreMesh(
        core_axis_name="c", subcore_axis_name="s", num_cores=_NUM_CORES
    )

    @functools.partial(
        pl.kernel,
        out_type=jax.ShapeDtypeStruct((n, wide), jnp.float32),
        mesh=mesh,
        scratch_types=[
            pltpu.VMEM((n_per_w,), jnp.int32),
            tuple(pltpu.VMEM((_CHUNK, wide), jnp.float32) for _ in range(_NBUF)),
            tuple(pltpu.SemaphoreType.DMA for _ in range(_NBUF)),
            tuple(pltpu.SemaphoreType.DMA for _ in range(_NBUF)),
        ],
        compiler_params=pltpu.CompilerParams(needs_layout_passes=False),
    )
    def gather_kernel(idx_hbm, table_hbm, out_hbm, idx_all, rows, gsems, wsems):
        wid = lax.axis_index("s") * _NUM_CORES + lax.axis_index("c")
        base = wid * n_per_w

        pltpu.sync_copy(idx_hbm.at[pl.ds(base, n_per_w)], idx_all)

        def g_start(g, b):
            pltpu.async_copy(
                table_hbm.at[idx_all.at[pl.ds(g * _CHUNK, _CHUNK)]],
                rows[b], gsems[b])

        def g_wait(b):
            pltpu.make_async_copy(
                table_hbm.at[idx_all.at[pl.ds(0, _CHUNK)]],
                rows[b], gsems[b]).wait()

        def w_start(g, b):
            pltpu.async_copy(
                rows[b], out_hbm.at[pl.ds(base + g * _CHUNK, _CHUNK)],
                wsems[b])

        def w_wait(b):
            pltpu.make_async_copy(
                rows[b], out_hbm.at[pl.ds(base, _CHUNK)], wsems[b]).wait()

        for j in range(_NBUF - 1):
            g_start(j, j)

        def step(g, b):
            g_wait(b)
            pb = (b - 1) % _NBUF

            @pl.when(g >= 1)
            def _():
                w_wait(pb)

            @pl.when(g + _NBUF - 1 < n_chunks)
            def _():
                g_start(g + _NBUF - 1, pb)

            w_start(g, b)

        def outer(i, carry):
            for b in range(_NBUF):
                step(i * _NBUF + b, b)
            return carry

        lax.fori_loop(0, n_chunks // _NBUF, outer, 0)
        w_wait((n_chunks - 1) % _NBUF)

    return gather_kernel


def kernel(tokens, table):
    b, s = tokens.shape
    v, d = table.shape
    n = b * s
    idx = tokens.reshape(n).astype(jnp.int32)
    table_w = jnp.pad(table, ((0, 0), (0, d)))
    wide_out = _make_gather(n, v, d)(idx, table_w)
    return lax.slice(wide_out, (0, 0), (n, d)).reshape(b, s, d)
